# baseline (speedup 1.0000x reference)
import jax
import jax.numpy as jnp
from jax.experimental import pallas as pl
from jax.experimental.pallas import tpu as pltpu

T = 64
D = 768
H = 512
H2 = 2 * H
E = 64

GB = 8
NBLK = E // GB


def _moe_body(cnt_ref, x_ref, eidf_ref, gw_ref, gb_ref, w_ref,
              out_ref, scale_ref):
    j = pl.program_id(0)
    xv = x_ref[...]
    eid_col = eidf_ref[...]

    @pl.when(j == 0)
    def _():
        logits = jnp.dot(xv, gw_ref[...], preferred_element_type=jnp.float32)
        logits = logits + gb_ref[...]
        m = jnp.max(logits, axis=1, keepdims=True)
        p = jnp.exp(logits - m)
        probs = p / jnp.sum(p, axis=1, keepdims=True)
        cols = jax.lax.broadcasted_iota(jnp.int32, (T, E), 1).astype(jnp.float32)
        sel = (cols == eid_col).astype(jnp.float32)
        scale_ref[...] = jnp.sum(probs * sel, axis=1, keepdims=True)
        out_ref[...] = jnp.zeros((T, H), jnp.float32)

    for k in range(GB):
        e_idx = j * GB + k

        @pl.when(cnt_ref[e_idx] > 0)
        def _(k=k, e_idx=e_idx):
            mask = (eid_col == e_idx.astype(jnp.float32)).astype(jnp.float32)
            xk = xv * mask
            proj = jnp.dot(xk, w_ref[k], preferred_element_type=jnp.float32)
            a = proj[:, :H]
            b = proj[:, H:]
            out_ref[...] += jax.lax.logistic(a) * a * b

    @pl.when(j == NBLK - 1)
    def _():
        out_ref[...] *= scale_ref[...]


@jax.jit
def _moe_call(cnt, x, eidf, gw, gb2, ew):
    grid_spec = pltpu.PrefetchScalarGridSpec(
        num_scalar_prefetch=1,
        grid=(NBLK,),
        in_specs=[
            pl.BlockSpec((T, D), lambda j, *_: (0, 0)),
            pl.BlockSpec((T, 1), lambda j, *_: (0, 0)),
            pl.BlockSpec((D, E), lambda j, *_: (0, 0)),
            pl.BlockSpec((1, E), lambda j, *_: (0, 0)),
            pl.BlockSpec((GB, D, H2), lambda j, *_: (j, 0, 0)),
        ],
        out_specs=pl.BlockSpec((T, H), lambda j, *_: (0, 0)),
        scratch_shapes=[
            pltpu.VMEM((T, 1), jnp.float32),
        ],
    )
    return pl.pallas_call(
        _moe_body,
        grid_spec=grid_spec,
        out_shape=jax.ShapeDtypeStruct((T, H), jnp.float32),
        compiler_params=pltpu.CompilerParams(
            dimension_semantics=("arbitrary",),
        ),
    )(cnt, x, eidf, gw, gb2, ew)


def kernel(x, expert_indices, expert_weights, gate_w, gate_b):
    cnt = jnp.zeros((E,), jnp.int32).at[expert_indices].add(1)
    eidf = expert_indices.astype(jnp.float32).reshape(T, 1)
    gb2 = gate_b.reshape(1, E)
    return _moe_call(cnt, x, eidf, gate_w, gb2, expert_weights)


# speedup vs baseline: 1.0327x; 1.0327x over previous
import jax
import jax.numpy as jnp
from jax.experimental import pallas as pl
from jax.experimental.pallas import tpu as pltpu

T = 64
D = 768
H = 512
H2 = 2 * H
E = 64

GB = 8
NBLK = E // GB


def _moe_body(start_ref, cnt_ref, order_ref,
              x_ref, eidf_ref, gw_ref, gb_ref, w_ref,
              out_ref, scale_ref):
    j = pl.program_id(0)
    xv = x_ref[...]

    @pl.when(j == 0)
    def _():
        logits = jnp.dot(xv, gw_ref[...], preferred_element_type=jnp.float32)
        logits = logits + gb_ref[...]
        m = jnp.max(logits, axis=1, keepdims=True)
        p = jnp.exp(logits - m)
        probs = p / jnp.sum(p, axis=1, keepdims=True)
        cols = jax.lax.broadcasted_iota(jnp.int32, (T, E), 1).astype(jnp.float32)
        sel = (cols == eidf_ref[...]).astype(jnp.float32)
        scale_ref[...] = jnp.sum(probs * sel, axis=1, keepdims=True)
        out_ref[...] = jnp.zeros((T, H), jnp.float32)

    def expert_tiles(k):
        e_idx = j * GB + k
        s = start_ref[e_idx]
        c = cnt_ref[e_idx]
        w = w_ref[k]

        def tile_body(q, carry):
            base = s + q * 8
            rows_m = []
            cols_m = []
            for r in range(8):
                pos = base + r
                valid = pos < s + c
                t_r = order_ref[jnp.minimum(pos, T - 1)]
                it_row = jax.lax.broadcasted_iota(jnp.int32, (1, T), 1)
                it_col = jax.lax.broadcasted_iota(jnp.int32, (T, 1), 0)
                rows_m.append(jnp.where(valid, (it_row == t_r).astype(jnp.float32), 0.0))
                cols_m.append(jnp.where(valid, (it_col == t_r).astype(jnp.float32), 0.0))
            gat = jnp.concatenate(rows_m, axis=0)
            sca = jnp.concatenate(cols_m, axis=1)
            rows = jnp.dot(gat, xv, preferred_element_type=jnp.float32)
            proj = jnp.dot(rows, w, preferred_element_type=jnp.float32)
            a = proj[:, :H]
            b = proj[:, H:]
            g = jax.lax.logistic(a) * a * b
            out_ref[...] += jnp.dot(sca, g, preferred_element_type=jnp.float32)
            return carry

        ntiles = jax.lax.div(c + 7, 8)
        jax.lax.fori_loop(0, ntiles, tile_body, 0)

    for k in range(GB):
        expert_tiles(k)

    @pl.when(j == NBLK - 1)
    def _():
        out_ref[...] *= scale_ref[...]


@jax.jit
def _moe_call(start, cnt, order, x, eidf, gw, gb2, ew):
    grid_spec = pltpu.PrefetchScalarGridSpec(
        num_scalar_prefetch=3,
        grid=(NBLK,),
        in_specs=[
            pl.BlockSpec((T, D), lambda j, *_: (0, 0)),
            pl.BlockSpec((T, 1), lambda j, *_: (0, 0)),
            pl.BlockSpec((D, E), lambda j, *_: (0, 0)),
            pl.BlockSpec((1, E), lambda j, *_: (0, 0)),
            pl.BlockSpec((GB, D, H2), lambda j, *_: (j, 0, 0)),
        ],
        out_specs=pl.BlockSpec((T, H), lambda j, *_: (0, 0)),
        scratch_shapes=[
            pltpu.VMEM((T, 1), jnp.float32),
        ],
    )
    return pl.pallas_call(
        _moe_body,
        grid_spec=grid_spec,
        out_shape=jax.ShapeDtypeStruct((T, H), jnp.float32),
        compiler_params=pltpu.CompilerParams(
            dimension_semantics=("arbitrary",),
        ),
    )(start, cnt, order, x, eidf, gw, gb2, ew)


def _routing(expert_indices):
    order = jnp.argsort(expert_indices).astype(jnp.int32)
    sorted_eid = jnp.take(expert_indices, order)
    eids = jnp.arange(E, dtype=sorted_eid.dtype)
    start = jnp.searchsorted(sorted_eid, eids, side="left").astype(jnp.int32)
    end = jnp.searchsorted(sorted_eid, eids, side="right").astype(jnp.int32)
    return start, end - start, order


def kernel(x, expert_indices, expert_weights, gate_w, gate_b):
    start, cnt, order = _routing(expert_indices)
    eidf = expert_indices.astype(jnp.float32).reshape(T, 1)
    gb2 = gate_b.reshape(1, E)
    return _moe_call(start, cnt, order, x, eidf, gate_w, gb2, expert_weights)
